# baseline (speedup 1.0000x reference)
import functools

import jax
import jax.numpy as jnp
from jax import lax
from jax.experimental import pallas as pl
from jax.experimental.pallas import tpu as pltpu
from jax.experimental.pallas import tpu_sc as plsc

_D0, _D1, _D2 = 6, 7, 3
_EMB = 128
_TROWS = 128

_NC, _NS = 2, 16
_NW = _NC * _NS
_CHUNK = 128
_NBUF = 3


def _sc_bond_encoder(e0, e1, e2, W0, W1, W2):
    n = e0.shape[0]
    bpw = n // _NW
    nround = bpw // (_CHUNK * _NBUF)
    tail = bpw - nround * _CHUNK * _NBUF
    ngrp = bpw // 16
    rows_per_tile = _TROWS // _NS
    mesh = plsc.VectorSubcoreMesh(core_axis_name="c", subcore_axis_name="s")

    @functools.partial(
        pl.kernel,
        out_type=jax.ShapeDtypeStruct((n, _EMB), jnp.float32),
        mesh=mesh,
        scratch_types=[
            pltpu.VMEM((_D0, _EMB), jnp.float32),
            pltpu.VMEM((_D1, _EMB), jnp.float32),
            pltpu.VMEM((_D2, _EMB), jnp.float32),
            pltpu.VMEM((_EMB,), jnp.float32),
            pltpu.VMEM((bpw,), jnp.int32),
            pltpu.VMEM((bpw,), jnp.int32),
            pltpu.VMEM((bpw,), jnp.int32),
            pltpu.VMEM((bpw,), jnp.int32),
            pltpu.VMEM((_NBUF, _CHUNK, _EMB), jnp.float32),
            pltpu.VMEM_SHARED((_TROWS, _EMB), jnp.float32),
            pltpu.SemaphoreType.DMA,
            pltpu.SemaphoreType.DMA((_NBUF,)),
            pltpu.SemaphoreType.DMA((_NBUF,)),
        ],
    )
    def k(e0_hbm, e1_hbm, e2_hbm, w0_hbm, w1_hbm, w2_hbm, out_hbm,
          w0_v, w1_v, w2_v, trow_v, e0_v, e1_v, e2_v, idx_v, rows_v, t_sh,
          esem, gsem, ssem):
        s = lax.axis_index("s")
        wid = s * _NC + lax.axis_index("c")
        base = wid * bpw

        eh0 = pltpu.async_copy(e0_hbm.at[pl.ds(base, bpw)], e0_v, esem)
        eh1 = pltpu.async_copy(e1_hbm.at[pl.ds(base, bpw)], e1_v, esem)
        eh2 = pltpu.async_copy(e2_hbm.at[pl.ds(base, bpw)], e2_v, esem)

        pltpu.sync_copy(w0_hbm, w0_v)
        pltpu.sync_copy(w1_hbm, w1_v)
        pltpu.sync_copy(w2_hbm, w2_v)

        def build_row(rl, _):
            r = s * rows_per_tile + rl
            i0 = jnp.minimum(r // (_D1 * _D2), _D0 - 1)
            i1 = (r // _D2) % _D1
            i2 = r % _D2
            for c in range(_EMB // 16):
                sl = pl.ds(c * 16, 16)
                v = jnp.zeros((16,), jnp.float32)
                for t in range(_D0):
                    v = v + w0_v[t, sl] * (i0 == t).astype(jnp.float32)
                for t in range(_D1):
                    v = v + w1_v[t, sl] * (i1 == t).astype(jnp.float32)
                for t in range(_D2):
                    v = v + w2_v[t, sl] * (i2 == t).astype(jnp.float32)
                trow_v[sl] = v
            pltpu.sync_copy(trow_v, t_sh.at[r])
            return None

        lax.fori_loop(0, rows_per_tile, build_row, None)
        plsc.subcore_barrier()

        eh0.wait()
        eh1.wait()
        eh2.wait()

        grp_per_round = _CHUNK * _NBUF // 16

        def cidx_grp(g):
            sl = pl.ds(g * 16, 16)
            v = (jnp.clip(e0_v[sl], 0, _D0 - 1) * (_D1 * _D2)
                 + jnp.clip(e1_v[sl], 0, _D1 - 1) * _D2
                 + jnp.clip(e2_v[sl], 0, _D2 - 1))
            idx_v[sl] = v

        for gg in range(grp_per_round):
            cidx_grp(gg)
        for gg in range(tail // 16):
            cidx_grp(ngrp - 1 - gg)

        def ring_round(it, _):
            j0 = it * _NBUF
            handles = []
            for b in range(_NBUF):
                @pl.when(it > 0)
                def _(b=b):
                    pltpu.make_async_copy(
                        rows_v.at[b], out_hbm.at[pl.ds(0, _CHUNK)],
                        ssem.at[b]).wait()
                handles.append(pltpu.async_copy(
                    t_sh.at[idx_v.at[pl.ds((j0 + b) * _CHUNK, _CHUNK)]],
                    rows_v.at[b], gsem.at[b]))

            @pl.when(it < nround - 1)
            def _():
                g0 = (it + 1) * grp_per_round
                for gg in range(grp_per_round):
                    cidx_grp(g0 + gg)

            for b in range(_NBUF):
                handles[b].wait()
                pltpu.async_copy(
                    rows_v.at[b],
                    out_hbm.at[pl.ds(base + (j0 + b) * _CHUNK, _CHUNK)],
                    ssem.at[b])
            return None

        lax.fori_loop(0, nround, ring_round, None)

        for b in range(_NBUF):
            pltpu.make_async_copy(
                rows_v.at[b], out_hbm.at[pl.ds(0, _CHUNK)], ssem.at[b]).wait()

        if tail:
            gt = pltpu.async_copy(
                t_sh.at[idx_v.at[pl.ds(bpw - tail, tail)]],
                rows_v.at[0, pl.ds(0, tail)], gsem.at[0])
            gt.wait()
            pltpu.sync_copy(
                rows_v.at[0, pl.ds(0, tail)],
                out_hbm.at[pl.ds(base + bpw - tail, tail)])

    return k(e0, e1, e2, W0, W1, W2)


def kernel(edge_attr, W0, W1, W2):
    e0 = edge_attr[:, 0]
    e1 = edge_attr[:, 1]
    e2 = edge_attr[:, 2]
    return _sc_bond_encoder(e0, e1, e2, W0, W1, W2)


# speedup vs baseline: 1.0257x; 1.0257x over previous
import functools

import jax
import jax.numpy as jnp
from jax import lax
from jax.experimental import pallas as pl
from jax.experimental.pallas import tpu as pltpu
from jax.experimental.pallas import tpu_sc as plsc

_D0, _D1, _D2 = 6, 7, 3
_EMB = 128
_TROWS = 128

_NC, _NS = 2, 16
_NW = _NC * _NS
_CHUNK = 40
_NBUF = 10


def _sc_bond_encoder(e0, e1, e2, W0, W1, W2):
    n = e0.shape[0]
    bpw = n // _NW
    nround = bpw // (_CHUNK * _NBUF)
    ngrp = bpw // 16
    rows_per_tile = _TROWS // _NS
    mesh = plsc.VectorSubcoreMesh(core_axis_name="c", subcore_axis_name="s")

    @functools.partial(
        pl.kernel,
        out_type=jax.ShapeDtypeStruct((n, _EMB), jnp.float32),
        mesh=mesh,
        scratch_types=[
            pltpu.VMEM((_D0, _EMB), jnp.float32),
            pltpu.VMEM((_D1, _EMB), jnp.float32),
            pltpu.VMEM((_D2, _EMB), jnp.float32),
            pltpu.VMEM((_EMB,), jnp.float32),
            pltpu.VMEM((bpw,), jnp.int32),
            pltpu.VMEM((bpw,), jnp.int32),
            pltpu.VMEM((bpw,), jnp.int32),
            pltpu.VMEM((bpw,), jnp.int32),
            pltpu.VMEM((_NBUF, _CHUNK, _EMB), jnp.float32),
            pltpu.VMEM_SHARED((_TROWS, _EMB), jnp.float32),
            pltpu.SemaphoreType.DMA,
            pltpu.SemaphoreType.DMA((_NBUF,)),
            pltpu.SemaphoreType.DMA((_NBUF,)),
        ],
    )
    def k(e0_hbm, e1_hbm, e2_hbm, w0_hbm, w1_hbm, w2_hbm, out_hbm,
          w0_v, w1_v, w2_v, trow_v, e0_v, e1_v, e2_v, idx_v, rows_v, t_sh,
          esem, gsem, ssem):
        s = lax.axis_index("s")
        wid = s * _NC + lax.axis_index("c")
        base = wid * bpw

        eh0 = pltpu.async_copy(e0_hbm.at[pl.ds(base, bpw)], e0_v, esem)
        eh1 = pltpu.async_copy(e1_hbm.at[pl.ds(base, bpw)], e1_v, esem)
        eh2 = pltpu.async_copy(e2_hbm.at[pl.ds(base, bpw)], e2_v, esem)

        pltpu.sync_copy(w0_hbm, w0_v)
        pltpu.sync_copy(w1_hbm, w1_v)
        pltpu.sync_copy(w2_hbm, w2_v)

        def build_row(rl, _):
            r = s * rows_per_tile + rl
            i0 = jnp.minimum(r // (_D1 * _D2), _D0 - 1)
            i1 = (r // _D2) % _D1
            i2 = r % _D2
            for c in range(_EMB // 16):
                sl = pl.ds(c * 16, 16)
                v = jnp.zeros((16,), jnp.float32)
                for t in range(_D0):
                    v = v + w0_v[t, sl] * (i0 == t).astype(jnp.float32)
                for t in range(_D1):
                    v = v + w1_v[t, sl] * (i1 == t).astype(jnp.float32)
                for t in range(_D2):
                    v = v + w2_v[t, sl] * (i2 == t).astype(jnp.float32)
                trow_v[sl] = v
            pltpu.sync_copy(trow_v, t_sh.at[r])
            return None

        lax.fori_loop(0, rows_per_tile, build_row, None)
        plsc.subcore_barrier()

        eh0.wait()
        eh1.wait()
        eh2.wait()

        grp_per_round = _CHUNK * _NBUF // 16

        def cidx_grp(g):
            sl = pl.ds(g * 16, 16)
            v = (jnp.clip(e0_v[sl], 0, _D0 - 1) * (_D1 * _D2)
                 + jnp.clip(e1_v[sl], 0, _D1 - 1) * _D2
                 + jnp.clip(e2_v[sl], 0, _D2 - 1))
            idx_v[sl] = v

        for gg in range(grp_per_round):
            cidx_grp(gg)

        def ring_round(it, _):
            j0 = it * _NBUF
            handles = []
            for b in range(_NBUF):
                @pl.when(it > 0)
                def _(b=b):
                    pltpu.make_async_copy(
                        rows_v.at[b], out_hbm.at[pl.ds(0, _CHUNK)],
                        ssem.at[b]).wait()
                handles.append(pltpu.async_copy(
                    t_sh.at[idx_v.at[pl.ds((j0 + b) * _CHUNK, _CHUNK)]],
                    rows_v.at[b], gsem.at[b]))

            @pl.when(it < nround - 1)
            def _():
                g0 = (it + 1) * grp_per_round
                for gg in range(grp_per_round):
                    cidx_grp(g0 + gg)

            for b in range(_NBUF):
                handles[b].wait()
                pltpu.async_copy(
                    rows_v.at[b],
                    out_hbm.at[pl.ds(base + (j0 + b) * _CHUNK, _CHUNK)],
                    ssem.at[b])
            return None

        lax.fori_loop(0, nround, ring_round, None)

        for b in range(_NBUF):
            pltpu.make_async_copy(
                rows_v.at[b], out_hbm.at[pl.ds(0, _CHUNK)], ssem.at[b]).wait()

    return k(e0, e1, e2, W0, W1, W2)


def kernel(edge_attr, W0, W1, W2):
    e0 = edge_attr[:, 0]
    e1 = edge_attr[:, 1]
    e2 = edge_attr[:, 2]
    return _sc_bond_encoder(e0, e1, e2, W0, W1, W2)
